# Initial kernel scaffold; baseline (speedup 1.0000x reference)
import jax
import jax.numpy as jnp
from jax.experimental import pallas as pl


def kernel(user_feat, hi, edge_index, att1_w, att1_b, att2_w, att2_b, att3_w, att3_b, w_w, w_b):
    raise NotImplementedError("write your pallas kernel here")


# speedup vs baseline: 7.1496x; 7.1496x over previous
import functools

import jax
import jax.numpy as jnp
from jax import lax
from jax.experimental import pallas as pl
from jax.experimental.pallas import tpu as pltpu
from jax.experimental.pallas import tpu_sc as plsc

_NC = 2
_NS = 16
_NW = _NC * _NS
_C = 80
_LANES = 16


def _node_proj_body(x_ref, h_ref, w1a_ref, w1b_ref, b1_ref, a_ref, b_ref):
    a_ref[...] = (
        jnp.dot(x_ref[...], w1a_ref[...], preferred_element_type=jnp.float32)
        + b1_ref[...]
    )
    b_ref[...] = jnp.dot(h_ref[...], w1b_ref[...], preferred_element_type=jnp.float32)


def _mlp_body(e1_ref, w2t_ref, b2_ref, a3_ref, b3_ref, out_ref):
    x = jnp.maximum(e1_ref[...], 0.0)
    h2 = jnp.maximum(
        jnp.dot(x, w2t_ref[...], preferred_element_type=jnp.float32) + b2_ref[...],
        0.0,
    )
    s = jnp.sum(h2 * a3_ref[...], axis=1, keepdims=True) + b3_ref[...]
    out_ref[...] = jnp.exp(s)


def _finish_body(h0_ref, h1_ref, s0_ref, s1_ref, wt_ref, wb_ref, out_ref):
    s = s0_ref[...] + s1_ref[...]
    inv = jnp.where(s > 0.0, 1.0 / s, 0.0)
    h = (h0_ref[...] + h1_ref[...]) * inv
    out_ref[...] = (
        jnp.dot(h, wt_ref[...], preferred_element_type=jnp.float32) + wb_ref[...]
    )


def _edge_gather_body(nchunks, a_hbm, b_hbm, tr_hbm, te_hbm, out_hbm,
                      idx_tr, idx_te, rows_a, rows_b, sem_a, sem_b):
    wid = lax.axis_index("s") * _NC + lax.axis_index("c")
    base = wid * (nchunks * _C)

    def chunk(j, carry):
        off = base + j * _C
        pltpu.sync_copy(tr_hbm.at[pl.ds(off, _C)], idx_tr)
        pltpu.sync_copy(te_hbm.at[pl.ds(off, _C)], idx_te)
        cp_a = pltpu.async_copy(a_hbm.at[idx_tr], rows_a, sem_a)
        cp_b = pltpu.async_copy(b_hbm.at[idx_te], rows_b, sem_b)
        cp_a.wait()
        cp_b.wait()

        def row(i, c2):
            for v in range(128 // _LANES):
                sl = pl.ds(v * _LANES, _LANES)
                rows_a[i, sl] = rows_a[i, sl] + rows_b[i, sl]
            return c2

        lax.fori_loop(0, _C, row, 0)
        pltpu.sync_copy(rows_a, out_hbm.at[pl.ds(off, _C), :])
        return carry

    lax.fori_loop(0, nchunks, chunk, 0)


def _scatter_body(nchunks, n_nodes, hi_hbm, tr_hbm, te_hbm, ex_hbm,
                  hs_out, ss_out,
                  hs_sh, ss_sh, idx_tr, idx_te, rows, exbuf, zrows, zscal, sem):
    cid = lax.axis_index("c")
    sid = lax.axis_index("s")
    wid = sid * _NC + cid
    base = wid * (nchunks * _C)

    zr = zrows.shape[0]
    rows_per_cp = n_nodes // 10

    def zrow(i, c):
        for v in range(128 // _LANES):
            zrows[i, pl.ds(v * _LANES, _LANES)] = jnp.zeros((_LANES,), jnp.float32)
        return c
    lax.fori_loop(0, zr, zrow, 0)

    def zsc(i, c):
        zscal[pl.ds(i * _LANES, _LANES)] = jnp.zeros((_LANES,), jnp.float32)
        return c
    lax.fori_loop(0, zscal.shape[0] // _LANES, zsc, 0)

    @pl.when(sid < 10)
    def _zero_hs():
        for k in range(rows_per_cp // zr):
            pltpu.sync_copy(zrows,
                            hs_sh.at[pl.ds(sid * rows_per_cp + k * zr, zr), :])

    @pl.when(sid == 0)
    def _zero_ssum():
        zn = zscal.shape[0]
        for k in range(n_nodes // zn):
            pltpu.sync_copy(zscal, ss_sh.at[pl.ds(k * zn, zn)])

    plsc.subcore_barrier()

    def chunk(j, carry):
        off = base + j * _C
        pltpu.sync_copy(tr_hbm.at[pl.ds(off, _C)], idx_tr)
        pltpu.sync_copy(te_hbm.at[pl.ds(off, _C)], idx_te)
        pltpu.sync_copy(ex_hbm.at[pl.ds(off, _C)], exbuf)
        pltpu.async_copy(hi_hbm.at[idx_tr], rows, sem).wait()

        dn = lax.GatherDimensionNumbers(offset_dims=(), collapsed_slice_dims=(0,),
                                        start_index_map=(0,))

        def row(i, c2):
            g = (i // _LANES) * _LANES
            lane = i - g
            ex16 = exbuf[pl.ds(g, _LANES)]
            exv = lax.gather(ex16, jnp.full((_LANES, 1), lane, jnp.int32), dn,
                             (1,), mode=lax.GatherScatterMode.PROMISE_IN_BOUNDS)
            for v in range(128 // _LANES):
                sl = pl.ds(v * _LANES, _LANES)
                rows[i, sl] = rows[i, sl] * exv
            return c2

        lax.fori_loop(0, _C, row, 0)
        pltpu.sync_copy(rows, hs_sh.at[idx_te], add=True)
        pltpu.sync_copy(exbuf, ss_sh.at[idx_te], add=True)
        return carry

    lax.fori_loop(0, nchunks, chunk, 0)

    plsc.subcore_barrier()

    @pl.when(sid < 10)
    def _copy_out():
        r0 = sid * rows_per_cp
        pltpu.sync_copy(hs_sh.at[pl.ds(r0, rows_per_cp), :],
                        hs_out.at[cid, pl.ds(r0, rows_per_cp), :])
        pltpu.sync_copy(ss_sh.at[pl.ds(r0, rows_per_cp)], zscal)
        pltpu.sync_copy(zscal,
                        ss_out.at[pl.ds(cid * n_nodes + r0, rows_per_cp)])


def kernel(user_feat, hi, edge_index, att1_w, att1_b, att2_w, att2_b,
           att3_w, att3_b, w_w, w_b):
    n, d = user_feat.shape
    e = edge_index.shape[1]
    assert d == 128 and e % (_NW * _C) == 0 and n % _NS == 0 and n % 10 == 0

    trust = edge_index[0].astype(jnp.int32)
    trustee = edge_index[1].astype(jnp.int32)

    w1a_t = att1_w[:, :d].T
    w1b_t = att1_w[:, d:].T
    b1 = att1_b[None, :]
    w2t = att2_w.T
    b2 = att2_b[None, :]
    a3 = att3_w
    b3 = att3_b.reshape(1, 1)
    wwt = w_w.T
    wb = w_b[None, :]

    bn = 1000
    grid_n = n // bn
    f32 = jnp.float32
    a_tab, b_tab = pl.pallas_call(
        _node_proj_body,
        grid=(grid_n,),
        in_specs=[
            pl.BlockSpec((bn, d), lambda i: (i, 0)),
            pl.BlockSpec((bn, d), lambda i: (i, 0)),
            pl.BlockSpec((d, d), lambda i: (0, 0)),
            pl.BlockSpec((d, d), lambda i: (0, 0)),
            pl.BlockSpec((1, d), lambda i: (0, 0)),
        ],
        out_specs=[
            pl.BlockSpec((bn, d), lambda i: (i, 0)),
            pl.BlockSpec((bn, d), lambda i: (i, 0)),
        ],
        out_shape=[
            jax.ShapeDtypeStruct((n, d), f32),
            jax.ShapeDtypeStruct((n, d), f32),
        ],
    )(user_feat, hi, w1a_t, w1b_t, b1)

    nchunks = e // (_NW * _C)
    mesh = plsc.VectorSubcoreMesh(core_axis_name="c", subcore_axis_name="s",
                                  num_cores=_NC, num_subcores=_NS)
    e1 = pl.kernel(
        functools.partial(_edge_gather_body, nchunks),
        out_type=jax.ShapeDtypeStruct((e, d), f32),
        mesh=mesh,
        scratch_types=[
            pltpu.VMEM((_C,), jnp.int32),
            pltpu.VMEM((_C,), jnp.int32),
            pltpu.VMEM((_C, d), f32),
            pltpu.VMEM((_C, d), f32),
            pltpu.SemaphoreType.DMA,
            pltpu.SemaphoreType.DMA,
        ],
    )(a_tab, b_tab, trust, trustee)

    be = 2560
    grid_e = e // be
    ex = pl.pallas_call(
        _mlp_body,
        grid=(grid_e,),
        in_specs=[
            pl.BlockSpec((be, d), lambda i: (i, 0)),
            pl.BlockSpec((d, d), lambda i: (0, 0)),
            pl.BlockSpec((1, d), lambda i: (0, 0)),
            pl.BlockSpec((1, d), lambda i: (0, 0)),
            pl.BlockSpec((1, 1), lambda i: (0, 0)),
        ],
        out_specs=pl.BlockSpec((be, 1), lambda i: (i, 0)),
        out_shape=jax.ShapeDtypeStruct((e, 1), f32),
    )(e1, w2t, b2, a3, b3)
    ex_flat = ex.reshape(e)

    hs_parts, ss_parts = pl.kernel(
        functools.partial(_scatter_body, nchunks, n),
        out_type=(
            jax.ShapeDtypeStruct((_NC, n, d), f32),
            jax.ShapeDtypeStruct((_NC * n,), f32),
        ),
        mesh=mesh,
        scratch_types=[
            pltpu.VMEM_SHARED((n, d), f32),
            pltpu.VMEM_SHARED((n,), f32),
            pltpu.VMEM((_C,), jnp.int32),
            pltpu.VMEM((_C,), jnp.int32),
            pltpu.VMEM((_C, d), f32),
            pltpu.VMEM((_C,), f32),
            pltpu.VMEM((200, d), f32),
            pltpu.VMEM((1000,), f32),
            pltpu.SemaphoreType.DMA,
        ],
    )(hi, trust, trustee, ex_flat)
    ss_parts = ss_parts.reshape(_NC, n)

    out = pl.pallas_call(
        _finish_body,
        grid=(grid_n,),
        in_specs=[
            pl.BlockSpec((bn, d), lambda i: (i, 0)),
            pl.BlockSpec((bn, d), lambda i: (i, 0)),
            pl.BlockSpec((bn, 1), lambda i: (i, 0)),
            pl.BlockSpec((bn, 1), lambda i: (i, 0)),
            pl.BlockSpec((d, d), lambda i: (0, 0)),
            pl.BlockSpec((1, d), lambda i: (0, 0)),
        ],
        out_specs=pl.BlockSpec((bn, d), lambda i: (i, 0)),
        out_shape=jax.ShapeDtypeStruct((n, d), f32),
    )(hs_parts[0], hs_parts[1], ss_parts[0][:, None], ss_parts[1][:, None],
      wwt, wb)
    return out
